# baseline (speedup 1.0000x reference)
import jax
import jax.numpy as jnp
from jax.experimental import pallas as pl
from jax.experimental.pallas import tpu as pltpu

_E = 8
_C = 96
_B = 64
_T = 4
_HW = 1024
_N = _B * _HW
_EPS = 1e-5
_BT = 8
_NSTEPS = _B // _BT


def _router_kernel(x0_ref, x1_ref, x2_ref, x3_ref, w_ref, g_ref, bt_ref,
                   tw_ref, ti_ref, lg_ref,
                   s_ref, a2_ref):
    j = pl.program_id(0)
    w = w_ref[:, :]
    y2sum = None
    for i in range(_BT):
        xs = ((x0_ref[i] + x1_ref[i])
              + (x2_ref[i] + x3_ref[i]))
        y = jax.lax.dot_general(
            w, xs, (((1,), (0,)), ((), ())),
            preferred_element_type=jnp.float32)
        srow = jax.lax.dot_general(
            jnp.ones((1, _HW), jnp.float32), y, (((1,), (1,)), ((), ())),
            preferred_element_type=jnp.float32)
        s_ref[pl.ds(j * _BT + i, 1), :] = srow
        y2 = y * y
        y2sum = y2 if y2sum is None else y2sum + y2

    @pl.when(j == 0)
    def _init():
        a2_ref[:, :] = y2sum

    @pl.when(j > 0)
    def _acc():
        a2_ref[:, :] = a2_ref[:, :] + y2sum

    @pl.when(j == _NSTEPS - 1)
    def _epilogue():
        s_pre = s_ref[:, :] * (1.0 / _T)
        mu = jnp.sum(s_pre, axis=0, keepdims=True) * (1.0 / _N)
        ey2_col = jnp.sum(a2_ref[:, :], axis=1, keepdims=True) * (
            1.0 / (_N * _T * _T))
        eye = (jax.lax.broadcasted_iota(jnp.int32, (_E, _E), 0)
               == jax.lax.broadcasted_iota(jnp.int32, (_E, _E), 1)
               ).astype(jnp.float32)
        ey2 = jnp.sum(ey2_col * eye, axis=0, keepdims=True)
        var = ey2 - mu * mu
        inv = jax.lax.rsqrt(var + _EPS)
        logits = g_ref[:, :] * (s_pre * (1.0 / _HW) - mu) * inv + bt_ref[:, :]
        lg_ref[:, :] = logits

        mx = jnp.max(logits, axis=1, keepdims=True)
        ex = jnp.exp(logits - mx)
        wsm = ex / jnp.sum(ex, axis=1, keepdims=True)

        iota = jax.lax.broadcasted_iota(jnp.int32, (_B, _E), 1)
        m1 = jnp.max(wsm, axis=1, keepdims=True)
        i1 = jnp.min(jnp.where(wsm >= m1, iota, _E), axis=1, keepdims=True)
        wm2 = jnp.where(iota == i1, -1e30, wsm)
        m2 = jnp.max(wm2, axis=1, keepdims=True)
        i2 = jnp.min(jnp.where(wm2 >= m2, iota, _E), axis=1, keepdims=True)
        io2 = jax.lax.broadcasted_iota(jnp.int32, (_B, 2), 1)
        tw_ref[:, :] = jnp.where(io2 == 0, m1, m2) / (m1 + m2)
        ti_ref[:, :] = jnp.where(io2 == 0, i1, i2)


def kernel(x, conv_w, conv_b, bn_gamma, bn_beta):
    t, bsz, c, h, w = x.shape
    x4 = x.reshape(t, bsz, c, h * w)
    xt = [x4[k] for k in range(t)]
    g2 = bn_gamma.reshape(1, _E).astype(jnp.float32)
    bt2 = bn_beta.reshape(1, _E).astype(jnp.float32)

    out = pl.pallas_call(
        _router_kernel,
        grid=(_NSTEPS,),
        in_specs=[
            pl.BlockSpec((_BT, c, h * w), lambda b: (b, 0, 0)),
            pl.BlockSpec((_BT, c, h * w), lambda b: (b, 0, 0)),
            pl.BlockSpec((_BT, c, h * w), lambda b: (b, 0, 0)),
            pl.BlockSpec((_BT, c, h * w), lambda b: (b, 0, 0)),
            pl.BlockSpec((_E, _C), lambda b: (0, 0)),
            pl.BlockSpec((1, _E), lambda b: (0, 0)),
            pl.BlockSpec((1, _E), lambda b: (0, 0)),
        ],
        out_specs=[
            pl.BlockSpec((_B, 2), lambda b: (0, 0)),
            pl.BlockSpec((_B, 2), lambda b: (0, 0)),
            pl.BlockSpec((_B, _E), lambda b: (0, 0)),
        ],
        out_shape=[
            jax.ShapeDtypeStruct((_B, 2), jnp.float32),
            jax.ShapeDtypeStruct((_B, 2), jnp.int32),
            jax.ShapeDtypeStruct((_B, _E), jnp.float32),
        ],
        scratch_shapes=[
            pltpu.VMEM((_B, _E), jnp.float32),
            pltpu.VMEM((_E, _HW), jnp.float32),
        ],
        compiler_params=pltpu.CompilerParams(
            dimension_semantics=("arbitrary",)),
    )(xt[0], xt[1], xt[2], xt[3], conv_w, g2, bt2)
    return (out[0], out[1], out[2])


# speedup vs baseline: 1.6110x; 1.6110x over previous
import jax
import jax.numpy as jnp
from jax.experimental import pallas as pl
from jax.experimental.pallas import tpu as pltpu

_E = 8
_C = 96
_B = 64
_T = 4
_HW = 1024
_N = _B * _HW
_EPS = 1e-5
_BT = 8
_NSTEPS = _B // _BT


def _router_kernel(x_ref, w_ref, g_ref, bt_ref,
                   tw_ref, ti_ref, lg_ref,
                   s_ref, a2_ref):
    j = pl.program_id(0)
    w = w_ref[:, :]
    y2sum = None
    for i in range(_BT):
        xs = ((x_ref[0, i] + x_ref[1, i])
              + (x_ref[2, i] + x_ref[3, i]))
        y = jax.lax.dot_general(
            w, xs, (((1,), (0,)), ((), ())),
            preferred_element_type=jnp.float32)
        srow = jax.lax.dot_general(
            jnp.ones((1, _HW), jnp.float32), y, (((1,), (1,)), ((), ())),
            preferred_element_type=jnp.float32)
        s_ref[pl.ds(j * _BT + i, 1), :] = srow
        y2 = y * y
        y2sum = y2 if y2sum is None else y2sum + y2

    @pl.when(j == 0)
    def _init():
        a2_ref[:, :] = y2sum

    @pl.when(j > 0)
    def _acc():
        a2_ref[:, :] = a2_ref[:, :] + y2sum

    @pl.when(j == _NSTEPS - 1)
    def _epilogue():
        s_pre = s_ref[:, :] * (1.0 / _T)
        mu = jnp.sum(s_pre, axis=0, keepdims=True) * (1.0 / _N)
        ey2_col = jnp.sum(a2_ref[:, :], axis=1, keepdims=True) * (
            1.0 / (_N * _T * _T))
        eye = (jax.lax.broadcasted_iota(jnp.int32, (_E, _E), 0)
               == jax.lax.broadcasted_iota(jnp.int32, (_E, _E), 1)
               ).astype(jnp.float32)
        ey2 = jnp.sum(ey2_col * eye, axis=0, keepdims=True)
        var = ey2 - mu * mu
        inv = jax.lax.rsqrt(var + _EPS)
        logits = g_ref[:, :] * (s_pre * (1.0 / _HW) - mu) * inv + bt_ref[:, :]
        lg_ref[:, :] = logits

        mx = jnp.max(logits, axis=1, keepdims=True)
        ex = jnp.exp(logits - mx)
        wsm = ex / jnp.sum(ex, axis=1, keepdims=True)

        iota = jax.lax.broadcasted_iota(jnp.int32, (_B, _E), 1)
        m1 = jnp.max(wsm, axis=1, keepdims=True)
        i1 = jnp.min(jnp.where(wsm >= m1, iota, _E), axis=1, keepdims=True)
        wm2 = jnp.where(iota == i1, -1e30, wsm)
        m2 = jnp.max(wm2, axis=1, keepdims=True)
        i2 = jnp.min(jnp.where(wm2 >= m2, iota, _E), axis=1, keepdims=True)
        io2 = jax.lax.broadcasted_iota(jnp.int32, (_B, 2), 1)
        tw_ref[:, :] = jnp.where(io2 == 0, m1, m2) / (m1 + m2)
        ti_ref[:, :] = jnp.where(io2 == 0, i1, i2)


def kernel(x, conv_w, conv_b, bn_gamma, bn_beta):
    t, bsz, c, h, w = x.shape
    x4 = x.reshape(t, bsz, c, h * w)
    g2 = bn_gamma.reshape(1, _E).astype(jnp.float32)
    bt2 = bn_beta.reshape(1, _E).astype(jnp.float32)

    out = pl.pallas_call(
        _router_kernel,
        grid=(_NSTEPS,),
        in_specs=[
            pl.BlockSpec((t, _BT, c, h * w), lambda b: (0, b, 0, 0)),
            pl.BlockSpec((_E, _C), lambda b: (0, 0)),
            pl.BlockSpec((1, _E), lambda b: (0, 0)),
            pl.BlockSpec((1, _E), lambda b: (0, 0)),
        ],
        out_specs=[
            pl.BlockSpec((_B, 2), lambda b: (0, 0)),
            pl.BlockSpec((_B, 2), lambda b: (0, 0)),
            pl.BlockSpec((_B, _E), lambda b: (0, 0)),
        ],
        out_shape=[
            jax.ShapeDtypeStruct((_B, 2), jnp.float32),
            jax.ShapeDtypeStruct((_B, 2), jnp.int32),
            jax.ShapeDtypeStruct((_B, _E), jnp.float32),
        ],
        scratch_shapes=[
            pltpu.VMEM((_B, _E), jnp.float32),
            pltpu.VMEM((_E, _HW), jnp.float32),
        ],
        compiler_params=pltpu.CompilerParams(
            dimension_semantics=("arbitrary",)),
    )(x4, conv_w, g2, bt2)
    return (out[0], out[1], out[2])
